# baseline (speedup 1.0000x reference)
import jax
import jax.numpy as jnp
from jax.experimental import pallas as pl
from jax.experimental.pallas import tpu as pltpu

ALPHA = 1.702
EPB = 2
NH = 4
NI = 4


def _mlp_kernel(x_ref, *refs):
    nw1 = EPB * NH
    nw2 = EPB * NI
    w1_refs = refs[:nw1]
    w2_refs = refs[nw1:nw1 + nw2]
    b1_ref, b2_ref, o_ref = refs[nw1 + nw2:]
    hb = w1_refs[0].shape[2]
    ib = w2_refs[0].shape[2]
    inter = w2_refs[0].shape[3]
    T = x_ref.shape[0]
    x = x_ref[...]
    for a in range(EPB):
        gu = b1_ref[0, a].astype(jnp.float32)
        for k in range(NH):
            gu = gu + jnp.dot(x[:, k * hb:(k + 1) * hb],
                              w1_refs[a * NH + k][0, 0],
                              preferred_element_type=jnp.float32)
        gate = gu[:, :inter]
        up = gu[:, inter:]
        glu = gate * jax.nn.sigmoid(gate * ALPHA)
        act = (up + 1.0) * glu
        out = b2_ref[0, a].astype(jnp.float32)
        for j in range(NI):
            out = out + jnp.dot(act[:, j * ib:(j + 1) * ib],
                                w2_refs[a * NI + j][0, 0],
                                preferred_element_type=jnp.float32)
        o_ref[a * T:(a + 1) * T, :] = out


def kernel(hidden_states, router_indices, routing_weights, gate_up_proj,
           gate_up_proj_bias, down_proj, down_proj_bias):
    del router_indices, routing_weights
    E, H, F2 = gate_up_proj.shape
    inter = down_proj.shape[1]
    T = hidden_states.shape[0]
    hb = H // NH
    ib = inter // NI
    EG = E // EPB
    w1v = gate_up_proj.reshape(EG, EPB * NH, hb, F2)
    w2v = down_proj.reshape(EG, EPB * NI, ib, H)
    b1 = gate_up_proj_bias.reshape(EG, EPB, 1, F2)
    b2 = down_proj_bias.reshape(EG, EPB, 1, H)
    in_specs = [pl.BlockSpec((T, H), lambda e: (0, 0))]
    operands = [hidden_states]
    for k in range(EPB * NH):
        in_specs.append(pl.BlockSpec((1, 1, hb, F2), lambda e, _k=k: (e, _k, 0, 0)))
        operands.append(w1v)
    for j in range(EPB * NI):
        in_specs.append(pl.BlockSpec((1, 1, ib, H), lambda e, _j=j: (e, _j, 0, 0)))
        operands.append(w2v)
    in_specs.append(pl.BlockSpec((1, EPB, 1, F2), lambda e: (e, 0, 0, 0)))
    operands.append(b1)
    in_specs.append(pl.BlockSpec((1, EPB, 1, H), lambda e: (e, 0, 0, 0)))
    operands.append(b2)
    out = pl.pallas_call(
        _mlp_kernel,
        grid=(EG,),
        in_specs=in_specs,
        out_specs=pl.BlockSpec((EPB * T, H), lambda e: (e, 0)),
        out_shape=jax.ShapeDtypeStruct((E * T, H), jnp.float32),
        compiler_params=pltpu.CompilerParams(
            dimension_semantics=("arbitrary",),
            vmem_limit_bytes=62 * 1024 * 1024,
        ),
    )(*operands)
    return out


# speedup vs baseline: 1.0215x; 1.0215x over previous
import jax
import jax.numpy as jnp
from jax.experimental import pallas as pl
from jax.experimental.pallas import tpu as pltpu

ALPHA = 1.702
NH = 4
NI = 4


def _mlp_kernel(x_ref, *refs):
    w1_refs = refs[:NH]
    w2_refs = refs[NH:NH + NI]
    b1_ref, b2_ref, o_ref = refs[NH + NI:]
    hb = w1_refs[0].shape[2]
    ib = w2_refs[0].shape[2]
    inter = w2_refs[0].shape[3]
    x = x_ref[...]
    gu = b1_ref[0].astype(jnp.float32)
    for k in range(NH):
        gu = gu + jnp.dot(x[:, k * hb:(k + 1) * hb], w1_refs[k][0, 0],
                          preferred_element_type=jnp.float32)
    gate = gu[:, :inter]
    up = gu[:, inter:]
    glu = gate * jax.nn.sigmoid(gate * ALPHA)
    act = (up + 1.0) * glu
    out = b2_ref[0].astype(jnp.float32)
    for j in range(NI):
        out = out + jnp.dot(act[:, j * ib:(j + 1) * ib], w2_refs[j][0, 0],
                            preferred_element_type=jnp.float32)
    o_ref[...] = out


def kernel(hidden_states, router_indices, routing_weights, gate_up_proj,
           gate_up_proj_bias, down_proj, down_proj_bias):
    del router_indices, routing_weights
    E, H, F2 = gate_up_proj.shape
    inter = down_proj.shape[1]
    T = hidden_states.shape[0]
    hb = H // NH
    ib = inter // NI
    w1v = gate_up_proj.reshape(E, NH, hb, F2)
    w2v = down_proj.reshape(E, NI, ib, H)
    b1 = gate_up_proj_bias.reshape(E, 1, F2)
    b2 = down_proj_bias.reshape(E, 1, H)
    in_specs = [pl.BlockSpec((T, H), lambda e: (0, 0))]
    operands = [hidden_states]
    for k in range(NH):
        in_specs.append(pl.BlockSpec((1, 1, hb, F2), lambda e, _k=k: (e, _k, 0, 0)))
        operands.append(w1v)
    for j in range(NI):
        in_specs.append(pl.BlockSpec((1, 1, ib, H), lambda e, _j=j: (e, _j, 0, 0)))
        operands.append(w2v)
    in_specs.append(pl.BlockSpec((1, 1, F2), lambda e: (e, 0, 0)))
    operands.append(b1)
    in_specs.append(pl.BlockSpec((1, 1, H), lambda e: (e, 0, 0)))
    operands.append(b2)
    out = pl.pallas_call(
        _mlp_kernel,
        grid=(E,),
        in_specs=in_specs,
        out_specs=pl.BlockSpec((T, H), lambda e: (e, 0)),
        out_shape=jax.ShapeDtypeStruct((E * T, H), jnp.float32),
        compiler_params=pltpu.CompilerParams(
            dimension_semantics=("arbitrary",),
        ),
    )(*operands)
    return out


# speedup vs baseline: 1.0243x; 1.0027x over previous
import jax
import jax.numpy as jnp
from jax.experimental import pallas as pl
from jax.experimental.pallas import tpu as pltpu

ALPHA = 1.702
NH = 4
NI = 4


def _mlp_kernel(x_ref, *refs):
    w1_refs = refs[:NH]
    w2_refs = refs[NH:NH + NI]
    b1_ref, b2_ref, o_ref = refs[NH + NI:]
    hb = w1_refs[0].shape[2]
    ib = w2_refs[0].shape[2]
    inter = w2_refs[0].shape[3]
    e = pl.program_id(0)
    x = x_ref[...]
    gu = b1_ref[e].astype(jnp.float32)
    for k in range(NH):
        gu = gu + jnp.dot(x[:, k * hb:(k + 1) * hb], w1_refs[k][0, 0],
                          preferred_element_type=jnp.float32)
    gate = gu[:, :inter]
    up = gu[:, inter:]
    glu = gate * jax.nn.sigmoid(gate * ALPHA)
    act = (up + 1.0) * glu
    out = b2_ref[e].astype(jnp.float32)
    for j in range(NI):
        out = out + jnp.dot(act[:, j * ib:(j + 1) * ib], w2_refs[j][0, 0],
                            preferred_element_type=jnp.float32)
    o_ref[...] = out


def kernel(hidden_states, router_indices, routing_weights, gate_up_proj,
           gate_up_proj_bias, down_proj, down_proj_bias):
    del router_indices, routing_weights
    E, H, F2 = gate_up_proj.shape
    inter = down_proj.shape[1]
    T = hidden_states.shape[0]
    hb = H // NH
    ib = inter // NI
    w1v = gate_up_proj.reshape(E, NH, hb, F2)
    w2v = down_proj.reshape(E, NI, ib, H)
    b1 = gate_up_proj_bias.reshape(E, 1, F2)
    b2 = down_proj_bias.reshape(E, 1, H)
    in_specs = [pl.BlockSpec((T, H), lambda e: (0, 0))]
    operands = [hidden_states]
    for k in range(NH):
        in_specs.append(pl.BlockSpec((1, 1, hb, F2), lambda e, _k=k: (e, _k, 0, 0)))
        operands.append(w1v)
    for j in range(NI):
        in_specs.append(pl.BlockSpec((1, 1, ib, H), lambda e, _j=j: (e, _j, 0, 0)))
        operands.append(w2v)
    in_specs.append(pl.BlockSpec((E, 1, F2), lambda e: (0, 0, 0)))
    operands.append(b1)
    in_specs.append(pl.BlockSpec((E, 1, H), lambda e: (0, 0, 0)))
    operands.append(b2)
    out = pl.pallas_call(
        _mlp_kernel,
        grid=(E,),
        in_specs=in_specs,
        out_specs=pl.BlockSpec((T, H), lambda e: (e, 0)),
        out_shape=jax.ShapeDtypeStruct((E * T, H), jnp.float32),
        compiler_params=pltpu.CompilerParams(
            dimension_semantics=("arbitrary",),
        ),
    )(*operands)
    return out
